# baseline (speedup 1.0000x reference)
import functools

import jax
import jax.numpy as jnp
from jax import lax
from jax.experimental import pallas as pl
from jax.experimental.pallas import tpu as pltpu
from jax.experimental.pallas import tpu_sc as plsc

D = 128
N = 100000
M = 5000

NW = 32
CW = 3136
NP = NW * CW
G16 = CW // 16
B = 112
NB = CW // B
MP = 5120
RPT = MP // 16


def _b_body(sp_ref, awt_ref, smol_ref):
    sp = sp_ref[...]
    col0 = (lax.broadcasted_iota(jnp.int32, (1, D), 1) == 0).astype(jnp.float32)
    w1m = awt_ref[:D, :] * col0
    s = lax.dot_general(sp, w1m, (((1,), (0,)), ((), ())),
                        preferred_element_type=jnp.float32)[:, 0]
    smol_ref[0, 0, :] = s


def _run_b(sp_pad, align_Wt):
    nblk = 4
    rows = MP // nblk
    return pl.pallas_call(
        _b_body,
        grid=(nblk,),
        in_specs=[
            pl.BlockSpec((rows, D), lambda i: (i, 0)),
            pl.BlockSpec((2 * D, 1), lambda i: (0, 0)),
        ],
        out_specs=[
            pl.BlockSpec((1, 1, rows), lambda i: (i, 0, 0)),
        ],
        out_shape=[
            jax.ShapeDtypeStruct((nblk, 1, rows), jnp.float32),
        ],
    )(sp_pad, align_Wt)


def _c_body(smol_hbm, w2b_hbm, mol_hbm, atom_hbm,
            e_hbm, dp_hbm, eap_hbm,
            smol_v, w2b_v, mol_v, e_v, denacc, rowbuf,
            idxblk, sem_in, sem_out, shared_ea):
    cid = lax.axis_index("c")
    sid = lax.axis_index("s")
    wid = sid * 2 + cid
    base = wid * CW
    iot = lax.iota(jnp.int32, 16)
    zeros16 = jnp.zeros((16,), jnp.float32)

    pltpu.sync_copy(smol_hbm, smol_v)
    pltpu.sync_copy(w2b_hbm, w2b_v)
    pltpu.sync_copy(mol_hbm.at[pl.ds(base, CW)], mol_v)

    def zrow(l, _):
        def zcol(j, _):
            denacc[l, pl.ds(j * 16, 16)] = zeros16
            return 0
        return lax.fori_loop(0, RPT, zcol, 0, unroll=4)
    lax.fori_loop(0, 2, zrow, 0)

    def zb(r, _):
        for k in range(8):
            rowbuf[0, r, pl.ds(k * 16, 16)] = zeros16
        return 0
    lax.fori_loop(0, 64, zb, 0, unroll=2)
    r0 = sid * RPT
    for k in range(5):
        pltpu.sync_copy(rowbuf.at[0, pl.ds(0, 64), :],
                        shared_ea.at[pl.ds(r0 + k * 64, 64), :])
    plsc.subcore_barrier()

    def rows_start(g):
        return jnp.minimum(base + g * B, N - B)

    def dma_desc(g, b):
        return pltpu.make_async_copy(atom_hbm.at[pl.ds(rows_start(g), B), :],
                                     rowbuf.at[b], sem_in.at[b])

    def scat_desc(b):
        return pltpu.make_async_copy(rowbuf.at[b], shared_ea.at[idxblk.at[b]],
                                     sem_out.at[b])

    for g0 in range(3):
        dma_desc(g0, g0).start()

    w2 = [w2b_v[pl.ds(16 * k, 16)] for k in range(8)]
    b_spl = w2b_v[pl.ds(D, 16)]
    lane0 = iot == 0
    lane_pair = [jnp.logical_and(iot >= 2 * p, iot < 2 * p + 2)
                 for p in range(8)]
    rows2 = lax.bitwise_and(iot, 1)

    def superiter(t, _):
        for b in range(4):
            g = 4 * t + b
            own_lo = base + g * B
            s_g = rows_start(g)
            lofs = s_g - base
            r_lo = own_lo - s_g
            for q in range(B // 16):
                idxblk[b, pl.ds(q * 16, 16)] = mol_v[pl.ds(lofs + q * 16, 16)]
            dma_desc(g, b).wait()

            @plsc.parallel_loop(0, B, 1, unroll=8)
            def rowloop(r):
                rv = [rowbuf[b, r, pl.ds(16 * k, 16)] for k in range(8)]
                p0 = rv[0] * w2[0]
                p1 = rv[1] * w2[1]
                p2 = rv[2] * w2[2]
                p3 = rv[3] * w2[3]
                p4 = rv[4] * w2[4]
                p5 = rv[5] * w2[5]
                p6 = rv[6] * w2[6]
                p7 = rv[7] * w2[7]
                acc = ((p0 + p1) + (p2 + p3)) + ((p4 + p5) + (p6 + p7))
                a_spl = jnp.full((16,), jnp.sum(acc), jnp.float32) + b_spl
                idxs = jnp.full((16,), lofs + r, jnp.int32)
                mol_spl = plsc.load_gather(mol_v, [idxs])
                s_spl = plsc.load_gather(smol_v, [mol_spl])
                t4 = s_spl + 4.0
                c_spl = jnp.where(t4 >= 0, t4, 0.01 * t4)
                sc = s_spl + a_spl
                sc = jnp.where(sc >= 0, sc, 0.01 * sc)
                e_spl = jnp.exp(sc - c_spl)
                own = r >= r_lo
                e_spl = jnp.where(own, e_spl, 0.0)
                plsc.store_scatter(e_v, [idxs], e_spl,
                                   mask=jnp.logical_and(lane0, own))
                for k in range(8):
                    rowbuf[b, r, pl.ds(16 * k, 16)] = rv[k] * e_spl

            for q in range(B // 16):
                m_own = iot + (16 * q) >= r_lo
                e16 = e_v[pl.ds(lofs + q * 16, 16)]
                mol16 = mol_v[pl.ds(lofs + q * 16, 16)]
                for p in range(8):
                    plsc.addupdate_scatter(
                        denacc, [rows2, mol16], e16,
                        mask=jnp.logical_and(m_own, lane_pair[p]))

            pltpu.async_copy(rowbuf.at[b], shared_ea.at[idxblk.at[b]],
                             sem_out.at[b], add=True)
            bp = (b + 3) % 4
            if b == 0:
                @pl.when(t > 0)
                def _():
                    scat_desc(bp).wait()
            else:
                scat_desc(bp).wait()

            @pl.when(g + 3 < NB)
            def _():
                dma_desc(g + 3, bp).start()
        return 0
    lax.fori_loop(0, NB // 4, superiter, 0)
    scat_desc((NB - 1) % 4).wait()
    pltpu.sync_copy(e_v, e_hbm.at[pl.ds(base, CW)])
    plsc.subcore_barrier()

    def dred(j, _):
        denacc[0, pl.ds(j * 16, 16)] = (denacc[0, pl.ds(j * 16, 16)]
                                        + denacc[1, pl.ds(j * 16, 16)])
        return 0
    lax.fori_loop(0, RPT, dred, 0, unroll=2)
    pltpu.sync_copy(denacc.at[0], dp_hbm.at[wid])

    for k in range(5):
        pltpu.sync_copy(shared_ea.at[pl.ds(r0 + k * 64, 64), :],
                        eap_hbm.at[cid, pl.ds(r0 + k * 64, 64), :])


def _run_c(smol, w2b, mol_pad, atom):
    mesh = plsc.VectorSubcoreMesh(core_axis_name="c", subcore_axis_name="s")
    f = functools.partial(
        pl.kernel,
        mesh=mesh,
        compiler_params=pltpu.CompilerParams(needs_layout_passes=False),
        out_type=(
            jax.ShapeDtypeStruct((NP,), jnp.float32),
            jax.ShapeDtypeStruct((NW, MP), jnp.float32),
            jax.ShapeDtypeStruct((2, MP, D), jnp.float32),
        ),
        scratch_types=[
            pltpu.VMEM((MP,), jnp.float32),
            pltpu.VMEM((D + 16,), jnp.float32),
            pltpu.VMEM((CW,), jnp.int32),
            pltpu.VMEM((CW,), jnp.float32),
            pltpu.VMEM((2, MP), jnp.float32),
            pltpu.VMEM((4, B, D), jnp.float32),
            pltpu.VMEM((4, B), jnp.int32),
            pltpu.SemaphoreType.DMA((4,)),
            pltpu.SemaphoreType.DMA((4,)),
            pltpu.VMEM_SHARED((MP, D), jnp.float32),
        ],
    )(_c_body)
    return f(smol, w2b, mol_pad, atom)


def _d1_body(dp_ref, den_ref, dinv_ref):
    den = jnp.sum(dp_ref[...], axis=0)
    den_ref[0, 0, :] = den
    dinv_ref[0, 0, :] = 1.0 / (den + 1e-8)


def _run_d1(dp):
    nblk = 4
    rows = MP // nblk
    return pl.pallas_call(
        _d1_body,
        grid=(nblk,),
        in_specs=[pl.BlockSpec((NW, rows), lambda i: (0, i))],
        out_specs=[
            pl.BlockSpec((1, 1, rows), lambda i: (i, 0, 0)),
            pl.BlockSpec((1, 1, rows), lambda i: (i, 0, 0)),
        ],
        out_shape=[
            jax.ShapeDtypeStruct((nblk, 1, rows), jnp.float32),
            jax.ShapeDtypeStruct((nblk, 1, rows), jnp.float32),
        ],
    )(dp)


def _d_body(eap_ref, den_ref, dinv_ref, sp_ref, attw_ref, attb_ref,
            gam_ref, bet_ref, mean_ref, var_ref, wih_ref, bih_ref,
            whh_ref, bhh_ref, upd_ref):
    den = den_ref[0, 0, :]
    dinv = dinv_ref[0, 0, :]
    ea = eap_ref[0] + eap_ref[1]
    wa = ea * dinv[:, None]
    wsum = den * dinv
    scale = gam_ref[...] / jnp.sqrt(var_ref[...] + 1e-6)
    bf = (attb_ref[...] - mean_ref[...]) * scale + bet_ref[...]
    ctx = lax.dot_general(wa, attw_ref[...], (((1,), (1,)), ((), ())),
                          preferred_element_type=jnp.float32) * scale
    ctx = ctx + wsum[:, None] * bf
    ctx = jnp.where(ctx > 0, ctx, jnp.exp(jnp.minimum(ctx, 0.0)) - 1.0)
    gi = lax.dot_general(ctx, wih_ref[...], (((1,), (1,)), ((), ())),
                         preferred_element_type=jnp.float32) + bih_ref[...]
    gh = lax.dot_general(sp_ref[...], whh_ref[...], (((1,), (1,)), ((), ())),
                         preferred_element_type=jnp.float32) + bhh_ref[...]
    r = jax.nn.sigmoid(gi[:, :D] + gh[:, :D])
    z = jax.nn.sigmoid(gi[:, D:2 * D] + gh[:, D:2 * D])
    n = jnp.tanh(gi[:, 2 * D:] + r * gh[:, 2 * D:])
    upd_ref[...] = (1.0 - z) * n + z * sp_ref[...]


def _run_d(eap, den4, dinv4, sp_pad, attend_W, attend_b, bn_gamma, bn_beta,
           bn_mean, bn_var, gru_w_ih, gru_b_ih, gru_w_hh, gru_b_hh):
    nblk = 4
    rows = MP // nblk
    return pl.pallas_call(
        _d_body,
        grid=(nblk,),
        in_specs=[
            pl.BlockSpec((2, rows, D), lambda i: (0, i, 0)),
            pl.BlockSpec((1, 1, rows), lambda i: (i, 0, 0)),
            pl.BlockSpec((1, 1, rows), lambda i: (i, 0, 0)),
            pl.BlockSpec((rows, D), lambda i: (i, 0)),
            pl.BlockSpec((D, D), lambda i: (0, 0)),
            pl.BlockSpec((1, D), lambda i: (0, 0)),
            pl.BlockSpec((1, D), lambda i: (0, 0)),
            pl.BlockSpec((1, D), lambda i: (0, 0)),
            pl.BlockSpec((1, D), lambda i: (0, 0)),
            pl.BlockSpec((1, D), lambda i: (0, 0)),
            pl.BlockSpec((3 * D, D), lambda i: (0, 0)),
            pl.BlockSpec((1, 3 * D), lambda i: (0, 0)),
            pl.BlockSpec((3 * D, D), lambda i: (0, 0)),
            pl.BlockSpec((1, 3 * D), lambda i: (0, 0)),
        ],
        out_specs=[
            pl.BlockSpec((rows, D), lambda i: (i, 0)),
        ],
        out_shape=[
            jax.ShapeDtypeStruct((MP, D), jnp.float32),
        ],
    )(eap, den4, dinv4, sp_pad, attend_W, attend_b, bn_gamma, bn_beta,
      bn_mean, bn_var, gru_w_ih, gru_b_ih, gru_w_hh, gru_b_hh)


def _e_body(e_hbm, mol_hbm, dinv_hbm, w_hbm, dinv_v, mol_v, e_v, w_v):
    cid = lax.axis_index("c")
    sid = lax.axis_index("s")
    wid = sid * 2 + cid
    base = wid * CW
    pltpu.sync_copy(dinv_hbm, dinv_v)
    pltpu.sync_copy(mol_hbm.at[pl.ds(base, CW)], mol_v)
    pltpu.sync_copy(e_hbm.at[pl.ds(base, CW)], e_v)

    def grp(j, _):
        mol16 = mol_v[pl.ds(j * 16, 16)]
        dv = plsc.load_gather(dinv_v, [mol16])
        w_v[pl.ds(j * 16, 16)] = e_v[pl.ds(j * 16, 16)] * dv
        return 0
    lax.fori_loop(0, G16, grp, 0)
    pltpu.sync_copy(w_v, w_hbm.at[pl.ds(base, CW)])


def _run_e(e_pad, mol_pad, dinv):
    mesh = plsc.VectorSubcoreMesh(core_axis_name="c", subcore_axis_name="s")
    f = functools.partial(
        pl.kernel,
        mesh=mesh,
        compiler_params=pltpu.CompilerParams(needs_layout_passes=False),
        out_type=jax.ShapeDtypeStruct((NP,), jnp.float32),
        scratch_types=[
            pltpu.VMEM((MP,), jnp.float32),
            pltpu.VMEM((CW,), jnp.int32),
            pltpu.VMEM((CW,), jnp.float32),
            pltpu.VMEM((CW,), jnp.float32),
        ],
    )(_e_body)
    return f(e_pad, mol_pad, dinv)


def kernel(superatom, atom, mol_index, align_W, align_b, attend_W, attend_b,
           bn_gamma, bn_beta, bn_mean, bn_var,
           gru_w_ih, gru_w_hh, gru_b_ih, gru_b_hh):
    mol32 = mol_index.astype(jnp.int32)
    mol_pad = jnp.pad(mol32, (0, NP - N))
    sp_pad = jnp.pad(superatom, ((0, MP - M), (0, 0)))
    align_Wt = align_W.T

    w2b = jnp.concatenate([align_W[0, D:],
                           jnp.broadcast_to(align_b.reshape(1), (16,))])

    (smol4,) = _run_b(sp_pad, align_Wt)
    smol = smol4.reshape(MP)

    e_pad, dp, eap = _run_c(smol, w2b, mol_pad, atom)

    den4, dinv4 = _run_d1(dp)

    (upd_p,) = _run_d(eap, den4, dinv4, sp_pad, attend_W,
                      attend_b.reshape(1, D), bn_gamma.reshape(1, D),
                      bn_beta.reshape(1, D), bn_mean.reshape(1, D),
                      bn_var.reshape(1, D), gru_w_ih,
                      gru_b_ih.reshape(1, 3 * D), gru_w_hh,
                      gru_b_hh.reshape(1, 3 * D))

    w_pad = _run_e(e_pad, mol_pad, dinv4.reshape(MP))

    return (upd_p[:M], w_pad[:N].reshape(N, 1))


# speedup vs baseline: 1.2714x; 1.2714x over previous
import functools

import jax
import jax.numpy as jnp
from jax import lax
from jax.experimental import pallas as pl
from jax.experimental.pallas import tpu as pltpu
from jax.experimental.pallas import tpu_sc as plsc

D = 128
N = 100000
M = 5000

NW = 32
CW = 3136
NP = NW * CW
G16 = CW // 16
B = 112
NB = CW // B
MP = 5120
RPT = MP // 16


def _b_body(sp_ref, awt_ref, smol_ref):
    sp = sp_ref[...]
    col0 = (lax.broadcasted_iota(jnp.int32, (1, D), 1) == 0).astype(jnp.float32)
    w1m = awt_ref[:D, :] * col0
    s = lax.dot_general(sp, w1m, (((1,), (0,)), ((), ())),
                        preferred_element_type=jnp.float32)[:, 0]
    smol_ref[0, 0, :] = s


def _run_b(sp_pad, align_Wt):
    nblk = 4
    rows = MP // nblk
    return pl.pallas_call(
        _b_body,
        grid=(nblk,),
        in_specs=[
            pl.BlockSpec((rows, D), lambda i: (i, 0)),
            pl.BlockSpec((2 * D, 1), lambda i: (0, 0)),
        ],
        out_specs=[
            pl.BlockSpec((1, 1, rows), lambda i: (i, 0, 0)),
        ],
        out_shape=[
            jax.ShapeDtypeStruct((nblk, 1, rows), jnp.float32),
        ],
    )(sp_pad, align_Wt)


def _c_body(smol_hbm, w2b_hbm, mol_hbm, atom_hbm,
            e_hbm, dp_hbm, eap_hbm,
            smol_v, w2b_v, mol_v, e_v, denacc, rowbuf,
            idxblk, sem_in, sem_out, shared_ea):
    cid = lax.axis_index("c")
    sid = lax.axis_index("s")
    wid = sid * 2 + cid
    base = wid * CW
    base_c = jnp.minimum(base, N - CW)
    iot = lax.iota(jnp.int32, 16)
    zeros16 = jnp.zeros((16,), jnp.float32)

    pltpu.sync_copy(smol_hbm, smol_v)
    pltpu.sync_copy(w2b_hbm, w2b_v)
    pltpu.sync_copy(mol_hbm.at[pl.ds(base_c, CW)], mol_v)

    def zrow(l, _):
        def zcol(j, _):
            denacc[l, pl.ds(j * 16, 16)] = zeros16
            return 0
        return lax.fori_loop(0, RPT, zcol, 0, unroll=4)
    lax.fori_loop(0, 2, zrow, 0)

    def zb(r, _):
        for k in range(8):
            rowbuf[0, r, pl.ds(k * 16, 16)] = zeros16
        return 0
    lax.fori_loop(0, 64, zb, 0, unroll=2)
    r0 = sid * RPT
    for k in range(5):
        pltpu.sync_copy(rowbuf.at[0, pl.ds(0, 64), :],
                        shared_ea.at[pl.ds(r0 + k * 64, 64), :])
    plsc.subcore_barrier()

    def rows_start(g):
        return jnp.minimum(base + g * B, N - B)

    def dma_desc(g, b):
        return pltpu.make_async_copy(atom_hbm.at[pl.ds(rows_start(g), B), :],
                                     rowbuf.at[b], sem_in.at[b])

    def scat_desc(b):
        return pltpu.make_async_copy(rowbuf.at[b], shared_ea.at[idxblk.at[b]],
                                     sem_out.at[b])

    for g0 in range(3):
        dma_desc(g0, g0).start()

    w2 = [w2b_v[pl.ds(16 * k, 16)] for k in range(8)]
    b_spl = w2b_v[pl.ds(D, 16)]
    lane0 = iot == 0
    lane_pair = [jnp.logical_and(iot >= 2 * p, iot < 2 * p + 2)
                 for p in range(8)]
    rows2 = lax.bitwise_and(iot, 1)

    def superiter(t, _):
        for b in range(4):
            g = 4 * t + b
            own_lo = base + g * B
            s_g = rows_start(g)
            lofs = s_g - base_c
            r_lo = own_lo - s_g
            for q in range(B // 16):
                idxblk[b, pl.ds(q * 16, 16)] = mol_v[pl.ds(lofs + q * 16, 16)]
            dma_desc(g, b).wait()

            @plsc.parallel_loop(0, B, 1, unroll=4)
            def rowloop(r):
                rv = [rowbuf[b, r, pl.ds(16 * k, 16)] for k in range(8)]
                p0 = rv[0] * w2[0]
                p1 = rv[1] * w2[1]
                p2 = rv[2] * w2[2]
                p3 = rv[3] * w2[3]
                p4 = rv[4] * w2[4]
                p5 = rv[5] * w2[5]
                p6 = rv[6] * w2[6]
                p7 = rv[7] * w2[7]
                acc = ((p0 + p1) + (p2 + p3)) + ((p4 + p5) + (p6 + p7))
                a_spl = jnp.full((16,), jnp.sum(acc), jnp.float32) + b_spl
                idxs = jnp.full((16,), lofs + r, jnp.int32)
                mol_spl = plsc.load_gather(mol_v, [idxs])
                s_spl = plsc.load_gather(smol_v, [mol_spl])
                t4 = s_spl + 4.0
                c_spl = jnp.where(t4 >= 0, t4, 0.01 * t4)
                sc = s_spl + a_spl
                sc = jnp.where(sc >= 0, sc, 0.01 * sc)
                e_spl = jnp.exp(sc - c_spl)
                own = r >= r_lo
                e_spl = jnp.where(own, e_spl, 0.0)
                plsc.store_scatter(e_v, [idxs], e_spl,
                                   mask=jnp.logical_and(lane0, own))
                for k in range(8):
                    rowbuf[b, r, pl.ds(16 * k, 16)] = rv[k] * e_spl

            for q in range(B // 16):
                m_own = iot + (16 * q) >= r_lo
                e16 = e_v[pl.ds(lofs + q * 16, 16)]
                mol16 = mol_v[pl.ds(lofs + q * 16, 16)]
                for p in range(8):
                    plsc.addupdate_scatter(
                        denacc, [rows2, mol16], e16,
                        mask=jnp.logical_and(m_own, lane_pair[p]))

            pltpu.async_copy(rowbuf.at[b], shared_ea.at[idxblk.at[b]],
                             sem_out.at[b], add=True)
            bp = (b + 3) % 4
            if b == 0:
                @pl.when(t > 0)
                def _():
                    scat_desc(bp).wait()
            else:
                scat_desc(bp).wait()

            @pl.when(g + 3 < NB)
            def _():
                dma_desc(g + 3, bp).start()
        return 0
    lax.fori_loop(0, NB // 4, superiter, 0)
    scat_desc((NB - 1) % 4).wait()
    ofs = base - base_c

    @pl.when(ofs == 0)
    def _():
        pltpu.sync_copy(e_v, e_hbm.at[pl.ds(base_c, CW)])

    @pl.when(ofs != 0)
    def _():
        pltpu.sync_copy(e_v.at[pl.ds(NP - N, CW - (NP - N))],
                        e_hbm.at[pl.ds(N - CW + (NP - N), CW - (NP - N))])
    plsc.subcore_barrier()

    def dred(j, _):
        denacc[0, pl.ds(j * 16, 16)] = (denacc[0, pl.ds(j * 16, 16)]
                                        + denacc[1, pl.ds(j * 16, 16)])
        return 0
    lax.fori_loop(0, RPT, dred, 0, unroll=2)
    pltpu.sync_copy(denacc.at[0], dp_hbm.at[wid])

    for k in range(5):
        pltpu.sync_copy(shared_ea.at[pl.ds(r0 + k * 64, 64), :],
                        eap_hbm.at[cid, pl.ds(r0 + k * 64, 64), :])


def _run_c(smol, w2b, mol_pad, atom):
    mesh = plsc.VectorSubcoreMesh(core_axis_name="c", subcore_axis_name="s")
    f = functools.partial(
        pl.kernel,
        mesh=mesh,
        compiler_params=pltpu.CompilerParams(needs_layout_passes=False),
        out_type=(
            jax.ShapeDtypeStruct((N,), jnp.float32),
            jax.ShapeDtypeStruct((NW, MP), jnp.float32),
            jax.ShapeDtypeStruct((2, MP, D), jnp.float32),
        ),
        scratch_types=[
            pltpu.VMEM((MP,), jnp.float32),
            pltpu.VMEM((D + 16,), jnp.float32),
            pltpu.VMEM((CW,), jnp.int32),
            pltpu.VMEM((CW,), jnp.float32),
            pltpu.VMEM((2, MP), jnp.float32),
            pltpu.VMEM((4, B, D), jnp.float32),
            pltpu.VMEM((4, B), jnp.int32),
            pltpu.SemaphoreType.DMA((4,)),
            pltpu.SemaphoreType.DMA((4,)),
            pltpu.VMEM_SHARED((MP, D), jnp.float32),
        ],
    )(_c_body)
    return f(smol, w2b, mol_pad, atom)


def _d1_body(dp_ref, den_ref, dinv_ref):
    den = jnp.sum(dp_ref[...], axis=0)
    den_ref[0, 0, :] = den
    dinv_ref[0, 0, :] = 1.0 / (den + 1e-8)


def _run_d1(dp):
    nblk = 4
    rows = MP // nblk
    return pl.pallas_call(
        _d1_body,
        grid=(nblk,),
        in_specs=[pl.BlockSpec((NW, rows), lambda i: (0, i))],
        out_specs=[
            pl.BlockSpec((1, 1, rows), lambda i: (i, 0, 0)),
            pl.BlockSpec((1, 1, rows), lambda i: (i, 0, 0)),
        ],
        out_shape=[
            jax.ShapeDtypeStruct((nblk, 1, rows), jnp.float32),
            jax.ShapeDtypeStruct((nblk, 1, rows), jnp.float32),
        ],
    )(dp)


def _d_body(eap_ref, den_ref, dinv_ref, sp_ref, attw_ref, attb_ref,
            gam_ref, bet_ref, mean_ref, var_ref, wih_ref, bih_ref,
            whh_ref, bhh_ref, upd_ref):
    den = den_ref[0, 0, :]
    dinv = dinv_ref[0, 0, :]
    ea = eap_ref[0] + eap_ref[1]
    wa = ea * dinv[:, None]
    wsum = den * dinv
    scale = gam_ref[...] / jnp.sqrt(var_ref[...] + 1e-6)
    bf = (attb_ref[...] - mean_ref[...]) * scale + bet_ref[...]
    ctx = lax.dot_general(wa, attw_ref[...], (((1,), (1,)), ((), ())),
                          preferred_element_type=jnp.float32) * scale
    ctx = ctx + wsum[:, None] * bf
    ctx = jnp.where(ctx > 0, ctx, jnp.exp(jnp.minimum(ctx, 0.0)) - 1.0)
    gi = lax.dot_general(ctx, wih_ref[...], (((1,), (1,)), ((), ())),
                         preferred_element_type=jnp.float32) + bih_ref[...]
    gh = lax.dot_general(sp_ref[...], whh_ref[...], (((1,), (1,)), ((), ())),
                         preferred_element_type=jnp.float32) + bhh_ref[...]
    r = jax.nn.sigmoid(gi[:, :D] + gh[:, :D])
    z = jax.nn.sigmoid(gi[:, D:2 * D] + gh[:, D:2 * D])
    n = jnp.tanh(gi[:, 2 * D:] + r * gh[:, 2 * D:])
    upd_ref[...] = (1.0 - z) * n + z * sp_ref[...]


def _run_d(eap, den4, dinv4, sp_pad, attend_W, attend_b, bn_gamma, bn_beta,
           bn_mean, bn_var, gru_w_ih, gru_b_ih, gru_w_hh, gru_b_hh):
    nblk = 4
    rows = MP // nblk
    return pl.pallas_call(
        _d_body,
        grid=(nblk,),
        in_specs=[
            pl.BlockSpec((2, rows, D), lambda i: (0, i, 0)),
            pl.BlockSpec((1, 1, rows), lambda i: (i, 0, 0)),
            pl.BlockSpec((1, 1, rows), lambda i: (i, 0, 0)),
            pl.BlockSpec((rows, D), lambda i: (i, 0)),
            pl.BlockSpec((D, D), lambda i: (0, 0)),
            pl.BlockSpec((1, D), lambda i: (0, 0)),
            pl.BlockSpec((1, D), lambda i: (0, 0)),
            pl.BlockSpec((1, D), lambda i: (0, 0)),
            pl.BlockSpec((1, D), lambda i: (0, 0)),
            pl.BlockSpec((1, D), lambda i: (0, 0)),
            pl.BlockSpec((3 * D, D), lambda i: (0, 0)),
            pl.BlockSpec((1, 3 * D), lambda i: (0, 0)),
            pl.BlockSpec((3 * D, D), lambda i: (0, 0)),
            pl.BlockSpec((1, 3 * D), lambda i: (0, 0)),
        ],
        out_specs=[
            pl.BlockSpec((rows, D), lambda i: (i, 0)),
        ],
        out_shape=[
            jax.ShapeDtypeStruct((MP, D), jnp.float32),
        ],
    )(eap, den4, dinv4, sp_pad, attend_W, attend_b, bn_gamma, bn_beta,
      bn_mean, bn_var, gru_w_ih, gru_b_ih, gru_w_hh, gru_b_hh)


def _e_body(e_hbm, mol_hbm, dinv_hbm, w_hbm, dinv_v, mol_v, e_v, w_v):
    cid = lax.axis_index("c")
    sid = lax.axis_index("s")
    wid = sid * 2 + cid
    base = jnp.minimum(wid * CW, N - CW)
    pltpu.sync_copy(dinv_hbm, dinv_v)
    pltpu.sync_copy(mol_hbm.at[pl.ds(base, CW)], mol_v)
    pltpu.sync_copy(e_hbm.at[pl.ds(base, CW)], e_v)

    def grp(j, _):
        mol16 = mol_v[pl.ds(j * 16, 16)]
        dv = plsc.load_gather(dinv_v, [mol16])
        w_v[pl.ds(j * 16, 16)] = e_v[pl.ds(j * 16, 16)] * dv
        return 0
    lax.fori_loop(0, G16, grp, 0)
    pltpu.sync_copy(w_v, w_hbm.at[pl.ds(base, CW)])


def _run_e(e_pad, mol_pad, dinv):
    mesh = plsc.VectorSubcoreMesh(core_axis_name="c", subcore_axis_name="s")
    f = functools.partial(
        pl.kernel,
        mesh=mesh,
        compiler_params=pltpu.CompilerParams(needs_layout_passes=False),
        out_type=jax.ShapeDtypeStruct((N,), jnp.float32),
        scratch_types=[
            pltpu.VMEM((MP,), jnp.float32),
            pltpu.VMEM((CW,), jnp.int32),
            pltpu.VMEM((CW,), jnp.float32),
            pltpu.VMEM((CW,), jnp.float32),
        ],
    )(_e_body)
    return f(e_pad, mol_pad, dinv)


def kernel(superatom, atom, mol_index, align_W, align_b, attend_W, attend_b,
           bn_gamma, bn_beta, bn_mean, bn_var,
           gru_w_ih, gru_w_hh, gru_b_ih, gru_b_hh):
    mol32 = mol_index.astype(jnp.int32)
    sp_pad = jnp.pad(superatom, ((0, MP - M), (0, 0)))
    align_Wt = align_W.T

    w2b = jnp.concatenate([align_W[0, D:],
                           jnp.broadcast_to(align_b.reshape(1), (16,))])

    (smol4,) = _run_b(sp_pad, align_Wt)
    smol = smol4.reshape(MP)

    e_arr, dp, eap = _run_c(smol, w2b, mol32, atom)

    den4, dinv4 = _run_d1(dp)

    (upd_p,) = _run_d(eap, den4, dinv4, sp_pad, attend_W,
                      attend_b.reshape(1, D), bn_gamma.reshape(1, D),
                      bn_beta.reshape(1, D), bn_mean.reshape(1, D),
                      bn_var.reshape(1, D), gru_w_ih,
                      gru_b_ih.reshape(1, 3 * D), gru_w_hh,
                      gru_b_hh.reshape(1, 3 * D))

    w_arr = _run_e(e_arr, mol32, dinv4.reshape(MP))

    return (upd_p[:M], w_arr.reshape(N, 1))
